# baseline (speedup 1.0000x reference)
import functools

import jax
import jax.numpy as jnp
import numpy as np
from jax import lax
from jax.experimental import pallas as pl
from jax.experimental.pallas import tpu as pltpu
from jax.experimental.pallas import tpu_sc as plsc

_FIELD_DIMS = [100000] * 26
_EMBED_DIM = 16
_NUM_FIELDS = len(_FIELD_DIMS)

_NC, _NS, _L = 2, 16, 16
_NW = _NC * _NS

_BATCH = 16384
_B = _BATCH * _NUM_FIELDS
_B_PER_W = _B // _NW
_CHUNK = 1664
_N_CHUNKS = _B_PER_W // _CHUNK
_ADD_STEPS = _CHUNK // _L

_OFFSETS = np.concatenate(([0], np.cumsum(_FIELD_DIMS)[:-1])).astype(np.int32)
_PATTERN = np.tile(_OFFSETS, _CHUNK // _NUM_FIELDS)

_mesh = plsc.VectorSubcoreMesh(core_axis_name="c", subcore_axis_name="s")


@functools.partial(
    pl.kernel,
    out_type=jax.ShapeDtypeStruct((_B, _EMBED_DIM), jnp.float32),
    mesh=_mesh,
    scratch_types=[
        pltpu.VMEM((_CHUNK,), jnp.int32),
        pltpu.VMEM((_CHUNK,), jnp.int32),
        pltpu.VMEM((_CHUNK,), jnp.int32),
        pltpu.VMEM((_CHUNK, _EMBED_DIM), jnp.float32),
        pltpu.SemaphoreType.DMA,
    ],
    compiler_params=pltpu.CompilerParams(use_tc_tiling_on_sc=False),
)
def _sc_embed(x_hbm, pat_hbm, table_hbm, out_hbm,
              pat_v, raw_v, idx_v, rows_v, sem):
    wid = lax.axis_index("s") * _NC + lax.axis_index("c")
    base = wid * _B_PER_W
    pltpu.sync_copy(pat_hbm, pat_v)

    def do_chunk(c, carry):
        off = base + c * _CHUNK
        pltpu.sync_copy(x_hbm.at[pl.ds(off, _CHUNK)], raw_v)

        def add_body(i, carry2):
            s = pl.ds(i * _L, _L)
            idx_v[s] = raw_v[s] + pat_v[s]
            return carry2

        lax.fori_loop(0, _ADD_STEPS, add_body, 0, unroll=4)
        pltpu.async_copy(table_hbm.at[idx_v], rows_v, sem).wait()
        pltpu.sync_copy(rows_v, out_hbm.at[pl.ds(off, _CHUNK)])
        return carry

    lax.fori_loop(0, _N_CHUNKS, do_chunk, 0)


def kernel(x, table):
    x_flat = jnp.reshape(x.astype(jnp.int32), (_B,))
    pat = jnp.asarray(_PATTERN)
    out = _sc_embed(x_flat, pat, table)
    return jnp.reshape(out, (_BATCH, _NUM_FIELDS, _EMBED_DIM))


# speedup vs baseline: 3.4265x; 3.4265x over previous
import functools

import jax
import jax.numpy as jnp
import numpy as np
from jax import lax
from jax.experimental import pallas as pl
from jax.experimental.pallas import tpu as pltpu
from jax.experimental.pallas import tpu_sc as plsc

_FIELD_DIMS = [100000] * 26
_EMBED_DIM = 16
_NUM_FIELDS = len(_FIELD_DIMS)
_ROWS = sum(_FIELD_DIMS)
_FIELD = 100000

_NC, _NS, _L = 2, 16, 16
_BATCH = 16384
_B = _BATCH * _NUM_FIELDS
_KPW = _BATCH // _NS

_SEGW = 100352
_CW = 1024
_NFULL = 97
_SHORT_N = 896
_SHORT_L = 640
_FPC = _NUM_FIELDS // _NC
_TAIL_LO = (_ROWS // 128) * 128
_TAIL_N = _ROWS - _TAIL_LO

_mesh = plsc.VectorSubcoreMesh(core_axis_name="c", subcore_axis_name="s")


@functools.partial(
    pl.kernel,
    out_type=jax.ShapeDtypeStruct((_B * _EMBED_DIM,), jnp.float32),
    mesh=_mesh,
    scratch_types=[
        pltpu.VMEM_SHARED((_EMBED_DIM * _SEGW,), jnp.float32),
        pltpu.VMEM((_EMBED_DIM, _CW), jnp.float32),
        pltpu.VMEM((_EMBED_DIM, 128), jnp.float32),
        pltpu.VMEM((_KPW,), jnp.int32),
        pltpu.VMEM((_KPW,), jnp.int32),
        pltpu.VMEM((_KPW,), jnp.float32),
        pltpu.SemaphoreType.DMA,
    ],
    compiler_params=pltpu.CompilerParams(use_tc_tiling_on_sc=True),
)
def _sc_embed(xt_hbm, tableT_hbm, tailT_hbm, out_hbm,
              seg, tp_v, tail_v, raw_v, idx_v, val_v, sem):
    c = lax.axis_index("c")
    s = lax.axis_index("s")

    def do_field(fi, carry):
        f = c * _FPC + fi
        lo = ((f * _FIELD) >> 7) << 7
        delta = f * _FIELD - lo
        is_last = f == _NUM_FIELDS - 1

        for r in range(7):
            t = s + 16 * r

            @pl.when(t < _NFULL)
            def _():
                pltpu.sync_copy(
                    tableT_hbm.at[:, pl.ds(pl.multiple_of(lo + t * _CW, 128),
                                           _CW)],
                    tp_v)
                for j in range(_EMBED_DIM):
                    pltpu.sync_copy(
                        tp_v.at[j],
                        seg.at[pl.ds(pl.multiple_of(j * _SEGW + t * _CW, 8),
                                     _CW)])

            @pl.when(jnp.logical_and(t == _NFULL, jnp.logical_not(is_last)))
            def _():
                pltpu.sync_copy(
                    tableT_hbm.at[:, pl.ds(
                        pl.multiple_of(lo + _NFULL * _CW, 128), _SHORT_N)],
                    tp_v.at[:, pl.ds(0, _SHORT_N)])
                for j in range(_EMBED_DIM):
                    pltpu.sync_copy(
                        tp_v.at[j, pl.ds(0, _SHORT_N)],
                        seg.at[pl.ds(
                            pl.multiple_of(j * _SEGW + _NFULL * _CW, 8),
                            _SHORT_N)])

            @pl.when(jnp.logical_and(t == _NFULL, is_last))
            def _():
                pltpu.sync_copy(
                    tableT_hbm.at[:, pl.ds(
                        pl.multiple_of(lo + _NFULL * _CW, 128), _SHORT_L)],
                    tp_v.at[:, pl.ds(0, _SHORT_L)])
                for j in range(_EMBED_DIM):
                    pltpu.sync_copy(
                        tp_v.at[j, pl.ds(0, _SHORT_L)],
                        seg.at[pl.ds(
                            pl.multiple_of(j * _SEGW + _NFULL * _CW, 8),
                            _SHORT_L)])

        @pl.when(jnp.logical_and(is_last, s == 0))
        def _():
            pltpu.sync_copy(tailT_hbm, tail_v)
            for j in range(_EMBED_DIM):
                pltpu.sync_copy(
                    tail_v.at[j],
                    seg.at[pl.ds(
                        pl.multiple_of(j * _SEGW + (_TAIL_LO - lo), 8), 128)])

        plsc.subcore_barrier()

        pltpu.sync_copy(
            xt_hbm.at[pl.ds(pl.multiple_of(f * _BATCH + s * _KPW, 128),
                            _KPW)],
            raw_v)

        for j in range(_EMBED_DIM):
            def mk_idx(w, carry2, _j=j):
                sl = pl.ds(w * _L, _L)
                idx_v[sl] = raw_v[sl] + (delta + _j * _SEGW)
                return carry2

            lax.fori_loop(0, _KPW // _L, mk_idx, 0, unroll=4)
            pltpu.async_copy(seg.at[idx_v], val_v, sem).wait()
            pltpu.sync_copy(
                val_v,
                out_hbm.at[pl.ds(
                    pl.multiple_of(
                        (f * _EMBED_DIM + j) * _BATCH + s * _KPW, 128),
                    _KPW)])

        plsc.subcore_barrier()
        return carry

    lax.fori_loop(0, _FPC, do_field, 0)


def kernel(x, table):
    xt_flat = jnp.reshape(jnp.swapaxes(x.astype(jnp.int32), 0, 1), (_B,))
    tableT = jnp.swapaxes(table, 0, 1)
    tail_pad = jnp.concatenate(
        [table[_TAIL_LO:], jnp.zeros((128 - _TAIL_N, _EMBED_DIM),
                                     jnp.float32)])
    tailT = jnp.swapaxes(tail_pad, 0, 1)
    out = _sc_embed(xt_flat, tableT, tailT)
    o = jnp.reshape(out, (_NUM_FIELDS, _EMBED_DIM, _BATCH))
    return jnp.transpose(o, (2, 0, 1))


# speedup vs baseline: 3.8478x; 1.1229x over previous
import functools

import jax
import jax.numpy as jnp
import numpy as np
from jax import lax
from jax.experimental import pallas as pl
from jax.experimental.pallas import tpu as pltpu
from jax.experimental.pallas import tpu_sc as plsc

_FIELD_DIMS = [100000] * 26
_EMBED_DIM = 16
_NUM_FIELDS = len(_FIELD_DIMS)
_ROWS = sum(_FIELD_DIMS)
_FIELD = 100000

_NC, _NS, _L = 2, 16, 16
_BATCH = 16384
_B = _BATCH * _NUM_FIELDS
_KPW = _BATCH // _NS
_DH = 8
_HBLK = _KPW * _DH

_SEGW = 100352
_CW = 2048
_NFULL = 48
_SHORT_N = 1920
_SHORT_L = 1664
_FPC = _NUM_FIELDS // _NC
_TAIL_LO = (_ROWS // 128) * 128
_TAIL_N = _ROWS - _TAIL_LO

_mesh = plsc.VectorSubcoreMesh(core_axis_name="c", subcore_axis_name="s")


@functools.partial(
    pl.kernel,
    out_type=jax.ShapeDtypeStruct((_B * _EMBED_DIM,), jnp.float32),
    mesh=_mesh,
    scratch_types=[
        pltpu.VMEM_SHARED((_DH * _SEGW,), jnp.float32),
        pltpu.VMEM((_DH, _CW), jnp.float32),
        pltpu.VMEM((_DH, _SHORT_N), jnp.float32),
        pltpu.VMEM((_DH, 128), jnp.float32),
        pltpu.VMEM((_KPW,), jnp.int32),
        pltpu.VMEM((_HBLK,), jnp.int32),
        pltpu.VMEM((_HBLK,), jnp.float32),
        pltpu.SemaphoreType.DMA,
        pltpu.SemaphoreType.DMA,
    ],
    compiler_params=pltpu.CompilerParams(use_tc_tiling_on_sc=True),
)
def _sc_embed(xt_hbm, tableT_hbm, tailT_hbm, out_hbm,
              seg, tp_v, tps_v, tail_v, raw_v, idx_v, val_v, sem_st, sem_g):
    c = lax.axis_index("c")
    s = lax.axis_index("s")

    def do_field(fi, carry):
        f = c * _FPC + fi
        lo = ((f * _FIELD) >> 7) << 7
        delta = f * _FIELD - lo
        is_last = f == _NUM_FIELDS - 1

        pltpu.sync_copy(
            xt_hbm.at[pl.ds(pl.multiple_of(f * _BATCH + s * _KPW, 128),
                            _KPW)],
            raw_v)

        for h in range(2):
            for r in range(3):
                t = s + 16 * r

                pltpu.sync_copy(
                    tableT_hbm.at[pl.ds(_DH * h, _DH),
                                  pl.ds(pl.multiple_of(lo + t * _CW, 128),
                                        _CW)],
                    tp_v)
                for j in range(_DH):
                    pltpu.async_copy(
                        tp_v.at[j],
                        seg.at[pl.ds(pl.multiple_of(j * _SEGW + t * _CW, 8),
                                     _CW)],
                        sem_st)
                for j in range(_DH):
                    pltpu.make_async_copy(
                        tp_v.at[j],
                        seg.at[pl.ds(pl.multiple_of(j * _SEGW + t * _CW, 8),
                                     _CW)],
                        sem_st).wait()

            @pl.when(jnp.logical_and(s == 15, jnp.logical_not(is_last)))
            def _():
                pltpu.sync_copy(
                    tableT_hbm.at[pl.ds(_DH * h, _DH), pl.ds(
                        pl.multiple_of(lo + _NFULL * _CW, 128), _SHORT_N)],
                    tps_v)
                for j in range(_DH):
                    pltpu.sync_copy(
                        tps_v.at[j],
                        seg.at[pl.ds(
                            pl.multiple_of(j * _SEGW + _NFULL * _CW, 8),
                            _SHORT_N)])

            @pl.when(jnp.logical_and(s == 15, is_last))
            def _():
                pltpu.sync_copy(
                    tableT_hbm.at[pl.ds(_DH * h, _DH), pl.ds(
                        pl.multiple_of(lo + _NFULL * _CW, 128), _SHORT_L)],
                    tps_v.at[:, pl.ds(0, _SHORT_L)])
                for j in range(_DH):
                    pltpu.sync_copy(
                        tps_v.at[j, pl.ds(0, _SHORT_L)],
                        seg.at[pl.ds(
                            pl.multiple_of(j * _SEGW + _NFULL * _CW, 8),
                            _SHORT_L)])

            @pl.when(jnp.logical_and(is_last, s == 14))
            def _():
                pltpu.sync_copy(tailT_hbm.at[pl.ds(_DH * h, _DH)], tail_v)
                for j in range(_DH):
                    pltpu.sync_copy(
                        tail_v.at[j],
                        seg.at[pl.ds(
                            pl.multiple_of(j * _SEGW + (_TAIL_LO - lo), 8),
                            128)])

            plsc.subcore_barrier()

            for j in range(_DH):
                def mk_idx(w, carry2, _j=j):
                    sl = pl.ds(_j * _KPW + w * _L, _L)
                    slr = pl.ds(w * _L, _L)
                    idx_v[sl] = raw_v[slr] + (delta + _j * _SEGW)
                    return carry2

                lax.fori_loop(0, _KPW // _L, mk_idx, 0, unroll=4)

            pltpu.async_copy(seg.at[idx_v], val_v, sem_g).wait()
            pltpu.sync_copy(
                val_v,
                out_hbm.at[pl.ds(
                    pl.multiple_of(
                        f * _BATCH * _EMBED_DIM + s * _KPW * _EMBED_DIM
                        + h * _HBLK, 128),
                    _HBLK)])

            plsc.subcore_barrier()

        return carry

    lax.fori_loop(0, _FPC, do_field, 0)


def kernel(x, table):
    xt_flat = jnp.reshape(jnp.swapaxes(x.astype(jnp.int32), 0, 1), (_B,))
    tableT = jnp.swapaxes(table, 0, 1)
    tail_pad = jnp.concatenate(
        [table[_TAIL_LO:], jnp.zeros((128 - _TAIL_N, _EMBED_DIM),
                                     jnp.float32)])
    tailT = jnp.swapaxes(tail_pad, 0, 1)
    out = _sc_embed(xt_flat, tableT, tailT)
    o = jnp.reshape(out, (_NUM_FIELDS, _NS, _EMBED_DIM, _KPW))
    return jnp.reshape(jnp.transpose(o, (1, 3, 0, 2)),
                       (_BATCH, _NUM_FIELDS, _EMBED_DIM))


# speedup vs baseline: 4.5097x; 1.1720x over previous
import functools

import jax
import jax.numpy as jnp
import numpy as np
from jax import lax
from jax.experimental import pallas as pl
from jax.experimental.pallas import tpu as pltpu
from jax.experimental.pallas import tpu_sc as plsc

_FIELD_DIMS = [100000] * 26
_EMBED_DIM = 16
_NUM_FIELDS = len(_FIELD_DIMS)
_ROWS = sum(_FIELD_DIMS)
_FIELD = 100000

_NC, _NS, _L = 2, 16, 16
_BATCH = 16384
_B = _BATCH * _NUM_FIELDS
_KPW = _BATCH // _NS
_DH = 8
_HBLK = _KPW * _DH

_SEGW = 100352
_CW = 2048
_NFULL = 48
_SHORT_N = 1920
_SHORT_L = 1664
_FPC = _NUM_FIELDS // _NC
_TAIL_LO = (_ROWS // 128) * 128
_TAIL_N = _ROWS - _TAIL_LO

_mesh = plsc.VectorSubcoreMesh(core_axis_name="c", subcore_axis_name="s")


@functools.partial(
    pl.kernel,
    out_type=jax.ShapeDtypeStruct((_B * _EMBED_DIM,), jnp.float32),
    mesh=_mesh,
    scratch_types=[
        pltpu.VMEM_SHARED((_DH * _SEGW,), jnp.float32),
        pltpu.VMEM((_DH, _CW), jnp.float32),
        pltpu.VMEM((_DH, _CW), jnp.float32),
        pltpu.VMEM((_DH, _CW), jnp.float32),
        pltpu.VMEM((_DH, 128), jnp.float32),
        pltpu.VMEM((_KPW,), jnp.int32),
        pltpu.VMEM((_HBLK,), jnp.int32),
        pltpu.VMEM((_HBLK,), jnp.float32),
        pltpu.SemaphoreType.DMA,
        pltpu.SemaphoreType.DMA,
        pltpu.SemaphoreType.DMA,
        pltpu.SemaphoreType.DMA,
    ],
    compiler_params=pltpu.CompilerParams(use_tc_tiling_on_sc=True),
)
def _sc_embed(xt_hbm, tableT_hbm, tailT_hbm, out_hbm,
              seg, tp0_v, tp1_v, tp2_v, tail_v, raw_v, idx_v, val_v,
              sem_ld, sem_st, sem_g, sem_out):
    c = lax.axis_index("c")
    s = lax.axis_index("s")

    def do_field(fi, carry):
        f = c * _FPC + fi
        lo = ((f * _FIELD) >> 7) << 7
        delta = f * _FIELD - lo
        is_last = f == _NUM_FIELDS - 1

        pltpu.sync_copy(
            xt_hbm.at[pl.ds(pl.multiple_of(f * _BATCH + s * _KPW, 128),
                            _KPW)],
            raw_v)

        for h in range(2):
            tps = (tp0_v, tp1_v, tp2_v)
            lds = []
            for r in range(3):
                t = s + 16 * r
                lds.append(pltpu.async_copy(
                    tableT_hbm.at[pl.ds(_DH * h, _DH),
                                  pl.ds(pl.multiple_of(lo + t * _CW, 128),
                                        _CW)],
                    tps[r], sem_ld))
            for r in range(3):
                t = s + 16 * r
                lds[r].wait()
                for j in range(_DH):
                    pltpu.async_copy(
                        tps[r].at[j],
                        seg.at[pl.ds(pl.multiple_of(j * _SEGW + t * _CW, 8),
                                     _CW)],
                        sem_st)
            for r in range(3):
                t = s + 16 * r
                for j in range(_DH):
                    pltpu.make_async_copy(
                        tps[r].at[j],
                        seg.at[pl.ds(pl.multiple_of(j * _SEGW + t * _CW, 8),
                                     _CW)],
                        sem_st).wait()

            @pl.when(jnp.logical_and(s == 15, jnp.logical_not(is_last)))
            def _():
                pltpu.sync_copy(
                    tableT_hbm.at[pl.ds(_DH * h, _DH), pl.ds(
                        pl.multiple_of(lo + _NFULL * _CW, 128), _SHORT_N)],
                    tp0_v.at[:, pl.ds(0, _SHORT_N)])
                for j in range(_DH):
                    pltpu.sync_copy(
                        tp0_v.at[j, pl.ds(0, _SHORT_N)],
                        seg.at[pl.ds(
                            pl.multiple_of(j * _SEGW + _NFULL * _CW, 8),
                            _SHORT_N)])

            @pl.when(jnp.logical_and(s == 15, is_last))
            def _():
                pltpu.sync_copy(
                    tableT_hbm.at[pl.ds(_DH * h, _DH), pl.ds(
                        pl.multiple_of(lo + _NFULL * _CW, 128), _SHORT_L)],
                    tp0_v.at[:, pl.ds(0, _SHORT_L)])
                for j in range(_DH):
                    pltpu.sync_copy(
                        tp0_v.at[j, pl.ds(0, _SHORT_L)],
                        seg.at[pl.ds(
                            pl.multiple_of(j * _SEGW + _NFULL * _CW, 8),
                            _SHORT_L)])

            @pl.when(jnp.logical_and(is_last, s == 14))
            def _():
                pltpu.sync_copy(tailT_hbm.at[pl.ds(_DH * h, _DH)], tail_v)
                for j in range(_DH):
                    pltpu.sync_copy(
                        tail_v.at[j],
                        seg.at[pl.ds(
                            pl.multiple_of(j * _SEGW + (_TAIL_LO - lo), 8),
                            128)])

            plsc.subcore_barrier()

            for j in range(_DH):
                def mk_idx(w, carry2, _j=j):
                    sl = pl.ds(_j * _KPW + w * _L, _L)
                    slr = pl.ds(w * _L, _L)
                    idx_v[sl] = raw_v[slr] + (delta + _j * _SEGW)
                    return carry2

                lax.fori_loop(0, _KPW // _L, mk_idx, 0, unroll=4)

            obase = (f * _BATCH * _EMBED_DIM + s * _KPW * _EMBED_DIM
                     + h * _HBLK)
            half = _HBLK // 2
            g0 = pltpu.async_copy(seg.at[idx_v.at[pl.ds(0, half)]],
                                  val_v.at[pl.ds(0, half)], sem_g)
            g1 = pltpu.async_copy(seg.at[idx_v.at[pl.ds(half, half)]],
                                  val_v.at[pl.ds(half, half)], sem_g)
            g0.wait()
            pltpu.async_copy(
                val_v.at[pl.ds(0, half)],
                out_hbm.at[pl.ds(pl.multiple_of(obase, 128), half)],
                sem_out)
            g1.wait()
            pltpu.async_copy(
                val_v.at[pl.ds(half, half)],
                out_hbm.at[pl.ds(pl.multiple_of(obase + half, 128), half)],
                sem_out)
            for q in range(2):
                pltpu.make_async_copy(
                    val_v.at[pl.ds(0, half)],
                    out_hbm.at[pl.ds(pl.multiple_of(obase, 128), half)],
                    sem_out).wait()

            plsc.subcore_barrier()

        return carry

    lax.fori_loop(0, _FPC, do_field, 0)


def kernel(x, table):
    xt_flat = jnp.reshape(jnp.swapaxes(x.astype(jnp.int32), 0, 1), (_B,))
    tableT = jnp.swapaxes(table, 0, 1)
    tail_pad = jnp.concatenate(
        [table[_TAIL_LO:], jnp.zeros((128 - _TAIL_N, _EMBED_DIM),
                                     jnp.float32)])
    tailT = jnp.swapaxes(tail_pad, 0, 1)
    out = _sc_embed(xt_flat, tableT, tailT)
    o = jnp.reshape(out, (_NUM_FIELDS, _NS, _EMBED_DIM, _KPW))
    return jnp.reshape(jnp.transpose(o, (1, 3, 0, 2)),
                       (_BATCH, _NUM_FIELDS, _EMBED_DIM))


# speedup vs baseline: 5.1025x; 1.1315x over previous
import functools

import jax
import jax.numpy as jnp
import numpy as np
from jax import lax
from jax.experimental import pallas as pl
from jax.experimental.pallas import tpu as pltpu
from jax.experimental.pallas import tpu_sc as plsc

_FIELD_DIMS = [100000] * 26
_EMBED_DIM = 16
_NUM_FIELDS = len(_FIELD_DIMS)
_ROWS = sum(_FIELD_DIMS)
_FIELD = 100000

_NC, _NS, _L = 2, 16, 16
_BATCH = 16384
_B = _BATCH * _NUM_FIELDS
_KPW = _BATCH // _NS
_DH = 8
_HBLK = _KPW * _DH

_SEGW = 100352
_CW = 2048
_NFULL = 48
_SHORT_N = 1920
_SHORT_L = 1664
_FPC = _NUM_FIELDS // _NC
_TAIL_LO = (_ROWS // 128) * 128
_TAIL_N = _ROWS - _TAIL_LO

_mesh = plsc.VectorSubcoreMesh(core_axis_name="c", subcore_axis_name="s")


@functools.partial(
    pl.kernel,
    out_type=jax.ShapeDtypeStruct((_B * _EMBED_DIM,), jnp.float32),
    mesh=_mesh,
    scratch_types=[
        pltpu.VMEM_SHARED((_DH * _SEGW,), jnp.float32),
        pltpu.VMEM((_DH, _CW), jnp.float32),
        pltpu.VMEM((_DH, _CW), jnp.float32),
        pltpu.VMEM((_DH, _CW), jnp.float32),
        pltpu.VMEM((_DH, 128), jnp.float32),
        pltpu.VMEM((_KPW,), jnp.int32),
        pltpu.VMEM((_HBLK,), jnp.int32),
        pltpu.VMEM((_HBLK,), jnp.float32),
        pltpu.SemaphoreType.DMA,
        pltpu.SemaphoreType.DMA,
        pltpu.SemaphoreType.DMA,
        pltpu.SemaphoreType.DMA,
    ],
    compiler_params=pltpu.CompilerParams(use_tc_tiling_on_sc=True),
)
def _sc_embed(xt_hbm, tableT_hbm, tailT_hbm, out_hbm,
              seg, tp0_v, tp1_v, tp2_v, tail_v, raw_v, idx_v, val_v,
              sem_ld, sem_st, sem_g, sem_out):
    c = lax.axis_index("c")
    s = lax.axis_index("s")

    def do_field(fi, carry):
        f = c * _FPC + fi
        lo = ((f * _FIELD) >> 7) << 7
        delta = f * _FIELD - lo
        is_last = f == _NUM_FIELDS - 1

        pltpu.sync_copy(
            xt_hbm.at[pl.ds(pl.multiple_of(f * _BATCH + s * _KPW, 128),
                            _KPW)],
            raw_v)

        for h in range(2):
            tps = (tp0_v, tp1_v, tp2_v)
            lds = []
            for r in range(3):
                t = s + 16 * r
                lds.append(pltpu.async_copy(
                    tableT_hbm.at[pl.ds(_DH * h, _DH),
                                  pl.ds(pl.multiple_of(lo + t * _CW, 128),
                                        _CW)],
                    tps[r], sem_ld))
            for j in range(_DH):
                def mk_idx(w, carry2, _j=j):
                    sl = pl.ds(_j * _KPW + w * _L, _L)
                    slr = pl.ds(w * _L, _L)
                    idx_v[sl] = raw_v[slr] + (delta + _j * _SEGW)
                    return carry2

                lax.fori_loop(0, _KPW // _L, mk_idx, 0, unroll=4)

            for r in range(3):
                t = s + 16 * r
                lds[r].wait()
                for j in range(_DH):
                    pltpu.async_copy(
                        tps[r].at[j],
                        seg.at[pl.ds(pl.multiple_of(j * _SEGW + t * _CW, 8),
                                     _CW)],
                        sem_st)
            for r in range(3):
                t = s + 16 * r
                for j in range(_DH):
                    pltpu.make_async_copy(
                        tps[r].at[j],
                        seg.at[pl.ds(pl.multiple_of(j * _SEGW + t * _CW, 8),
                                     _CW)],
                        sem_st).wait()

            @pl.when(jnp.logical_and(s == 15, jnp.logical_not(is_last)))
            def _():
                pltpu.sync_copy(
                    tableT_hbm.at[pl.ds(_DH * h, _DH), pl.ds(
                        pl.multiple_of(lo + _NFULL * _CW, 128), _SHORT_N)],
                    tp0_v.at[:, pl.ds(0, _SHORT_N)])
                for j in range(_DH):
                    pltpu.sync_copy(
                        tp0_v.at[j, pl.ds(0, _SHORT_N)],
                        seg.at[pl.ds(
                            pl.multiple_of(j * _SEGW + _NFULL * _CW, 8),
                            _SHORT_N)])

            @pl.when(jnp.logical_and(s == 15, is_last))
            def _():
                pltpu.sync_copy(
                    tableT_hbm.at[pl.ds(_DH * h, _DH), pl.ds(
                        pl.multiple_of(lo + _NFULL * _CW, 128), _SHORT_L)],
                    tp0_v.at[:, pl.ds(0, _SHORT_L)])
                for j in range(_DH):
                    pltpu.sync_copy(
                        tp0_v.at[j, pl.ds(0, _SHORT_L)],
                        seg.at[pl.ds(
                            pl.multiple_of(j * _SEGW + _NFULL * _CW, 8),
                            _SHORT_L)])

            @pl.when(jnp.logical_and(is_last, s == 14))
            def _():
                pltpu.sync_copy(tailT_hbm.at[pl.ds(_DH * h, _DH)], tail_v)
                for j in range(_DH):
                    pltpu.sync_copy(
                        tail_v.at[j],
                        seg.at[pl.ds(
                            pl.multiple_of(j * _SEGW + (_TAIL_LO - lo), 8),
                            128)])

            plsc.subcore_barrier()

            obase = (f * _BATCH * _EMBED_DIM + s * _KPW * _EMBED_DIM
                     + h * _HBLK)
            qn = _HBLK // 4
            gs = [pltpu.async_copy(seg.at[idx_v.at[pl.ds(q * qn, qn)]],
                                   val_v.at[pl.ds(q * qn, qn)], sem_g)
                  for q in range(4)]
            for q in range(4):
                gs[q].wait()
                pltpu.async_copy(
                    val_v.at[pl.ds(q * qn, qn)],
                    out_hbm.at[pl.ds(pl.multiple_of(obase + q * qn, 128),
                                     qn)],
                    sem_out)
            for q in range(4):
                pltpu.make_async_copy(
                    val_v.at[pl.ds(q * qn, qn)],
                    out_hbm.at[pl.ds(pl.multiple_of(obase + q * qn, 128),
                                     qn)],
                    sem_out).wait()

            plsc.subcore_barrier()

        return carry

    lax.fori_loop(0, _FPC, do_field, 0)


def kernel(x, table):
    xt_flat = jnp.reshape(jnp.swapaxes(x.astype(jnp.int32), 0, 1), (_B,))
    tableT = jnp.swapaxes(table, 0, 1)
    tail_pad = jnp.concatenate(
        [table[_TAIL_LO:], jnp.zeros((128 - _TAIL_N, _EMBED_DIM),
                                     jnp.float32)])
    tailT = jnp.swapaxes(tail_pad, 0, 1)
    out = _sc_embed(xt_flat, tableT, tailT)
    o = jnp.reshape(out, (_NUM_FIELDS, _NS, _EMBED_DIM, _KPW))
    return jnp.reshape(jnp.transpose(o, (1, 3, 0, 2)),
                       (_BATCH, _NUM_FIELDS, _EMBED_DIM))


# speedup vs baseline: 5.1211x; 1.0037x over previous
import functools

import jax
import jax.numpy as jnp
import numpy as np
from jax import lax
from jax.experimental import pallas as pl
from jax.experimental.pallas import tpu as pltpu
from jax.experimental.pallas import tpu_sc as plsc

_FIELD_DIMS = [100000] * 26
_EMBED_DIM = 16
_NUM_FIELDS = len(_FIELD_DIMS)
_ROWS = sum(_FIELD_DIMS)
_FIELD = 100000

_NC, _NS, _L = 2, 16, 16
_BATCH = 16384
_B = _BATCH * _NUM_FIELDS
_KPW = _BATCH // _NS
_DH = 8
_HBLK = _KPW * _DH

_SEGW = 100352
_CW = 2048
_NFULL = 48
_SHORT_N = 1920
_SHORT_L = 1664
_FPC = _NUM_FIELDS // _NC
_TAIL_LO = (_ROWS // 128) * 128
_TAIL_N = _ROWS - _TAIL_LO

_mesh = plsc.VectorSubcoreMesh(core_axis_name="c", subcore_axis_name="s")


@functools.partial(
    pl.kernel,
    out_type=jax.ShapeDtypeStruct((_B * _EMBED_DIM,), jnp.float32),
    mesh=_mesh,
    scratch_types=[
        pltpu.VMEM_SHARED((_DH * _SEGW,), jnp.float32),
        pltpu.VMEM((_DH, _CW), jnp.float32),
        pltpu.VMEM((_DH, _CW), jnp.float32),
        pltpu.VMEM((_DH, _CW), jnp.float32),
        pltpu.VMEM((_DH, 128), jnp.float32),
        pltpu.VMEM((_KPW,), jnp.int32),
        pltpu.VMEM((_HBLK,), jnp.int32),
        pltpu.VMEM((_HBLK,), jnp.float32),
        pltpu.SemaphoreType.DMA,
        pltpu.SemaphoreType.DMA,
        pltpu.SemaphoreType.DMA,
        pltpu.SemaphoreType.DMA,
    ],
    compiler_params=pltpu.CompilerParams(use_tc_tiling_on_sc=True),
)
def _sc_embed(xt_hbm, tableT_hbm, tailT_hbm, out_hbm,
              seg, tp0_v, tp1_v, tp2_v, tail_v, raw_v, idx_v, val_v,
              sem_ld, sem_st, sem_g, sem_out):
    c = lax.axis_index("c")
    s = lax.axis_index("s")

    def do_field(fi, carry):
        f = c * _FPC + fi
        lo = ((f * _FIELD) >> 7) << 7
        delta = f * _FIELD - lo
        is_last = f == _NUM_FIELDS - 1

        pltpu.sync_copy(
            xt_hbm.at[pl.ds(pl.multiple_of(f * _BATCH + s * _KPW, 128),
                            _KPW)],
            raw_v)

        for h in range(2):
            tps = (tp0_v, tp1_v, tp2_v)
            lds = []
            for r in range(3):
                t = s + 16 * r
                lds.append(pltpu.async_copy(
                    tableT_hbm.at[pl.ds(_DH * h, _DH),
                                  pl.ds(pl.multiple_of(lo + t * _CW, 128),
                                        _CW)],
                    tps[r], sem_ld))
            for j in range(_DH):
                def mk_idx(w, carry2, _j=j):
                    sl = pl.ds(_j * _KPW + w * _L, _L)
                    slr = pl.ds(w * _L, _L)
                    idx_v[sl] = raw_v[slr] + (delta + _j * _SEGW)
                    return carry2

                lax.fori_loop(0, _KPW // _L, mk_idx, 0, unroll=4)

            for r in range(3):
                t = s + 16 * r
                lds[r].wait()
                for j in range(_DH):
                    pltpu.async_copy(
                        tps[r].at[j],
                        seg.at[pl.ds(pl.multiple_of(j * _SEGW + t * _CW, 8),
                                     _CW)],
                        sem_st)
            for r in range(3):
                t = s + 16 * r
                for j in range(_DH):
                    pltpu.make_async_copy(
                        tps[r].at[j],
                        seg.at[pl.ds(pl.multiple_of(j * _SEGW + t * _CW, 8),
                                     _CW)],
                        sem_st).wait()

            @pl.when(jnp.logical_and(s == 15, jnp.logical_not(is_last)))
            def _():
                pltpu.sync_copy(
                    tableT_hbm.at[pl.ds(_DH * h, _DH), pl.ds(
                        pl.multiple_of(lo + _NFULL * _CW, 128), _SHORT_N)],
                    tp0_v.at[:, pl.ds(0, _SHORT_N)])
                for j in range(_DH):
                    pltpu.sync_copy(
                        tp0_v.at[j, pl.ds(0, _SHORT_N)],
                        seg.at[pl.ds(
                            pl.multiple_of(j * _SEGW + _NFULL * _CW, 8),
                            _SHORT_N)])

            @pl.when(jnp.logical_and(s == 15, is_last))
            def _():
                pltpu.sync_copy(
                    tableT_hbm.at[pl.ds(_DH * h, _DH), pl.ds(
                        pl.multiple_of(lo + _NFULL * _CW, 128), _SHORT_L)],
                    tp0_v.at[:, pl.ds(0, _SHORT_L)])
                for j in range(_DH):
                    pltpu.sync_copy(
                        tp0_v.at[j, pl.ds(0, _SHORT_L)],
                        seg.at[pl.ds(
                            pl.multiple_of(j * _SEGW + _NFULL * _CW, 8),
                            _SHORT_L)])

            @pl.when(jnp.logical_and(is_last, s == 14))
            def _():
                pltpu.sync_copy(tailT_hbm.at[pl.ds(_DH * h, _DH)], tail_v)
                for j in range(_DH):
                    pltpu.sync_copy(
                        tail_v.at[j],
                        seg.at[pl.ds(
                            pl.multiple_of(j * _SEGW + (_TAIL_LO - lo), 8),
                            128)])

            plsc.subcore_barrier()

            obase = (f * _BATCH * _EMBED_DIM + s * _KPW * _EMBED_DIM
                     + h * _HBLK)
            qn = _HBLK // 8
            gs = [pltpu.async_copy(seg.at[idx_v.at[pl.ds(q * qn, qn)]],
                                   val_v.at[pl.ds(q * qn, qn)], sem_g)
                  for q in range(8)]
            for q in range(8):
                gs[q].wait()
                pltpu.async_copy(
                    val_v.at[pl.ds(q * qn, qn)],
                    out_hbm.at[pl.ds(pl.multiple_of(obase + q * qn, 128),
                                     qn)],
                    sem_out)
            for q in range(8):
                pltpu.make_async_copy(
                    val_v.at[pl.ds(q * qn, qn)],
                    out_hbm.at[pl.ds(pl.multiple_of(obase + q * qn, 128),
                                     qn)],
                    sem_out).wait()

            plsc.subcore_barrier()

        return carry

    lax.fori_loop(0, _FPC, do_field, 0)


def kernel(x, table):
    xt_flat = jnp.reshape(jnp.swapaxes(x.astype(jnp.int32), 0, 1), (_B,))
    tableT = jnp.swapaxes(table, 0, 1)
    tail_pad = jnp.concatenate(
        [table[_TAIL_LO:], jnp.zeros((128 - _TAIL_N, _EMBED_DIM),
                                     jnp.float32)])
    tailT = jnp.swapaxes(tail_pad, 0, 1)
    out = _sc_embed(xt_flat, tableT, tailT)
    o = jnp.reshape(out, (_NUM_FIELDS, _NS, _EMBED_DIM, _KPW))
    return jnp.reshape(jnp.transpose(o, (1, 3, 0, 2)),
                       (_BATCH, _NUM_FIELDS, _EMBED_DIM))


# speedup vs baseline: 5.6381x; 1.1010x over previous
import functools

import jax
import jax.numpy as jnp
import numpy as np
from jax import lax
from jax.experimental import pallas as pl
from jax.experimental.pallas import tpu as pltpu
from jax.experimental.pallas import tpu_sc as plsc

_FIELD_DIMS = [100000] * 26
_EMBED_DIM = 16
_NUM_FIELDS = len(_FIELD_DIMS)
_ROWS = sum(_FIELD_DIMS)
_FIELD = 100000

_NC, _NS, _L = 2, 16, 16
_BATCH = 16384
_B = _BATCH * _NUM_FIELDS
_KPW = _BATCH // _NS
_DH = 8
_HBLK = _KPW * _DH

_SEGW = 100352
_CW = 2048
_NFULL = 48
_SHORT_N = 1920
_SHORT_L = 1664
_FPC = _NUM_FIELDS // _NC
_TAIL_LO = (_ROWS // 128) * 128
_TAIL_N = _ROWS - _TAIL_LO

_mesh = plsc.VectorSubcoreMesh(core_axis_name="c", subcore_axis_name="s")


@functools.partial(
    pl.kernel,
    out_type=jax.ShapeDtypeStruct((_B * _EMBED_DIM,), jnp.float32),
    mesh=_mesh,
    scratch_types=[
        pltpu.VMEM_SHARED((_DH * _SEGW,), jnp.float32),
        pltpu.VMEM((_DH, _CW), jnp.float32),
        pltpu.VMEM((_DH, _CW), jnp.float32),
        pltpu.VMEM((_DH, _CW), jnp.float32),
        pltpu.VMEM((_DH, 128), jnp.float32),
        pltpu.VMEM((_KPW,), jnp.int32),
        pltpu.VMEM((_HBLK,), jnp.int32),
        pltpu.VMEM((_HBLK,), jnp.float32),
        pltpu.SemaphoreType.DMA,
        pltpu.SemaphoreType.DMA,
        pltpu.SemaphoreType.DMA,
        pltpu.SemaphoreType.DMA,
    ],
    compiler_params=pltpu.CompilerParams(use_tc_tiling_on_sc=True),
)
def _sc_embed(xt_hbm, tableT_hbm, tailT_hbm, out_hbm,
              seg, tp0_v, tp1_v, tp2_v, tail_v, raw_v, idx_v, val_v,
              sem_ld, sem_st, sem_g, sem_out):
    c = lax.axis_index("c")
    s = lax.axis_index("s")

    def do_field(fi, carry):
        f = c * _FPC + fi
        lo = ((f * _FIELD) >> 7) << 7
        delta = f * _FIELD - lo
        is_last = f == _NUM_FIELDS - 1

        pltpu.sync_copy(
            xt_hbm.at[pl.ds(pl.multiple_of(f * _BATCH + s * _KPW, 128),
                            _KPW)],
            raw_v)

        tps = (tp0_v, tp1_v, tp2_v)

        def fire_loads(h):
            return [pltpu.async_copy(
                tableT_hbm.at[pl.ds(_DH * h, _DH),
                              pl.ds(pl.multiple_of(lo + (s + 16 * r) * _CW,
                                                   128), _CW)],
                tps[r], sem_ld) for r in range(3)]

        def do_stripes(h, lds):
            for r in range(3):
                t = s + 16 * r
                lds[r].wait()
                for j in range(_DH):
                    pltpu.async_copy(
                        tps[r].at[j],
                        seg.at[pl.ds(pl.multiple_of(j * _SEGW + t * _CW, 8),
                                     _CW)],
                        sem_st)
            for r in range(3):
                t = s + 16 * r
                for j in range(_DH):
                    pltpu.make_async_copy(
                        tps[r].at[j],
                        seg.at[pl.ds(pl.multiple_of(j * _SEGW + t * _CW, 8),
                                     _CW)],
                        sem_st).wait()

            @pl.when(jnp.logical_and(s == 15, jnp.logical_not(is_last)))
            def _():
                pltpu.sync_copy(
                    tableT_hbm.at[pl.ds(_DH * h, _DH), pl.ds(
                        pl.multiple_of(lo + _NFULL * _CW, 128), _SHORT_N)],
                    tp0_v.at[:, pl.ds(0, _SHORT_N)])
                for j in range(_DH):
                    pltpu.sync_copy(
                        tp0_v.at[j, pl.ds(0, _SHORT_N)],
                        seg.at[pl.ds(
                            pl.multiple_of(j * _SEGW + _NFULL * _CW, 8),
                            _SHORT_N)])

            @pl.when(jnp.logical_and(s == 15, is_last))
            def _():
                pltpu.sync_copy(
                    tableT_hbm.at[pl.ds(_DH * h, _DH), pl.ds(
                        pl.multiple_of(lo + _NFULL * _CW, 128), _SHORT_L)],
                    tp0_v.at[:, pl.ds(0, _SHORT_L)])
                for j in range(_DH):
                    pltpu.sync_copy(
                        tp0_v.at[j, pl.ds(0, _SHORT_L)],
                        seg.at[pl.ds(
                            pl.multiple_of(j * _SEGW + _NFULL * _CW, 8),
                            _SHORT_L)])

            @pl.when(jnp.logical_and(is_last, s == 14))
            def _():
                pltpu.sync_copy(tailT_hbm.at[pl.ds(_DH * h, _DH)], tail_v)
                for j in range(_DH):
                    pltpu.sync_copy(
                        tail_v.at[j],
                        seg.at[pl.ds(
                            pl.multiple_of(j * _SEGW + (_TAIL_LO - lo), 8),
                            128)])

        def extract(h):
            obase = (f * _BATCH * _EMBED_DIM + s * _KPW * _EMBED_DIM
                     + h * _HBLK)
            qn = _HBLK // 8
            gs = [pltpu.async_copy(seg.at[idx_v.at[pl.ds(q * qn, qn)]],
                                   val_v.at[pl.ds(q * qn, qn)], sem_g)
                  for q in range(8)]
            for q in range(8):
                gs[q].wait()
                pltpu.async_copy(
                    val_v.at[pl.ds(q * qn, qn)],
                    out_hbm.at[pl.ds(pl.multiple_of(obase + q * qn, 128),
                                     qn)],
                    sem_out)
            for q in range(8):
                pltpu.make_async_copy(
                    val_v.at[pl.ds(q * qn, qn)],
                    out_hbm.at[pl.ds(pl.multiple_of(obase + q * qn, 128),
                                     qn)],
                    sem_out).wait()

        lds0 = fire_loads(0)
        for j in range(_DH):
            def mk_idx(w, carry2, _j=j):
                sl = pl.ds(_j * _KPW + w * _L, _L)
                slr = pl.ds(w * _L, _L)
                idx_v[sl] = raw_v[slr] + (delta + _j * _SEGW)
                return carry2

            lax.fori_loop(0, _KPW // _L, mk_idx, 0, unroll=4)
        do_stripes(0, lds0)
        plsc.subcore_barrier()
        lds1 = fire_loads(1)
        extract(0)
        plsc.subcore_barrier()
        do_stripes(1, lds1)
        plsc.subcore_barrier()
        extract(1)
        plsc.subcore_barrier()

        return carry

    lax.fori_loop(0, _FPC, do_field, 0)


def kernel(x, table):
    xt_flat = jnp.reshape(jnp.swapaxes(x.astype(jnp.int32), 0, 1), (_B,))
    tableT = jnp.swapaxes(table, 0, 1)
    tail_pad = jnp.concatenate(
        [table[_TAIL_LO:], jnp.zeros((128 - _TAIL_N, _EMBED_DIM),
                                     jnp.float32)])
    tailT = jnp.swapaxes(tail_pad, 0, 1)
    out = _sc_embed(xt_flat, tableT, tailT)
    o = jnp.reshape(out, (_NUM_FIELDS, _NS, _EMBED_DIM, _KPW))
    return jnp.reshape(jnp.transpose(o, (1, 3, 0, 2)),
                       (_BATCH, _NUM_FIELDS, _EMBED_DIM))
